# baseline (device time: 30579 ns/iter reference)
import jax
import jax.numpy as jnp
from jax import lax
from jax.experimental import pallas as pl
from jax.experimental.pallas import tpu as pltpu


def kernel(x, router, W1, W2):
    t_loc, d = x.shape
    e_loc = W1.shape[0]

    def body(x_ref, r_ref, w1_ref, w2_ref, out_ref,
             x_other, r_other, p_send, p_recv, send_sems, recv_sems):
        my_x = lax.axis_index("x")
        my_y = lax.axis_index("y")
        my_z = lax.axis_index("z")
        partner = (my_x, 1 - my_y, my_z)

        barrier = pltpu.get_barrier_semaphore()
        pl.semaphore_signal(barrier, inc=1, device_id=partner,
                            device_id_type=pl.DeviceIdType.MESH)
        pl.semaphore_wait(barrier, 1)

        rdma_x = pltpu.make_async_remote_copy(
            src_ref=x_ref, dst_ref=x_other,
            send_sem=send_sems.at[0], recv_sem=recv_sems.at[0],
            device_id=partner, device_id_type=pl.DeviceIdType.MESH)
        rdma_r = pltpu.make_async_remote_copy(
            src_ref=r_ref, dst_ref=r_other,
            send_sem=send_sems.at[1], recv_sem=recv_sems.at[1],
            device_id=partner, device_id_type=pl.DeviceIdType.MESH)
        rdma_x.start()
        rdma_r.start()
        rdma_x.wait()
        rdma_r.wait()

        r_mine = r_ref[:, :]
        r_oth = r_other[:, :]

        def expert_partial(xh):
            g = jnp.concatenate(
                [jnp.dot(xh, r_mine, preferred_element_type=jnp.float32),
                 jnp.dot(xh, r_oth, preferred_element_type=jnp.float32)],
                axis=1)
            m1 = jnp.max(g, axis=1, keepdims=True)
            m2 = jnp.max(jnp.where(g < m1, g, -jnp.inf), axis=1,
                         keepdims=True)
            sel = g >= m2
            e = jnp.where(sel, jnp.exp(g - m1), 0.0)
            w = e / jnp.sum(e, axis=1, keepdims=True)
            acc = jnp.zeros((t_loc, d), dtype=jnp.float32)
            for j in range(e_loc):
                h = jnp.maximum(
                    jnp.dot(xh, w1_ref[j], preferred_element_type=jnp.float32),
                    0.0)
                pj = jnp.dot(h, w2_ref[j], preferred_element_type=jnp.float32)
                acc = acc + pj * w[:, j:j + 1]
            return acc

        p_mine = expert_partial(x_ref[:, :])
        p_send[:, :] = expert_partial(x_other[:, :])

        rdma_p = pltpu.make_async_remote_copy(
            src_ref=p_send, dst_ref=p_recv,
            send_sem=send_sems.at[2], recv_sem=recv_sems.at[2],
            device_id=partner, device_id_type=pl.DeviceIdType.MESH)
        rdma_p.start()
        rdma_p.wait()

        out_ref[:, :] = p_mine + p_recv[:, :]

    return pl.pallas_call(
        body,
        out_shape=jax.ShapeDtypeStruct((t_loc, d), jnp.float32),
        in_specs=[pl.BlockSpec(memory_space=pltpu.VMEM)] * 4,
        out_specs=pl.BlockSpec(memory_space=pltpu.VMEM),
        scratch_shapes=[
            pltpu.VMEM((t_loc, d), jnp.float32),
            pltpu.VMEM(router.shape, jnp.float32),
            pltpu.VMEM((t_loc, d), jnp.float32),
            pltpu.VMEM((t_loc, d), jnp.float32),
            pltpu.SemaphoreType.DMA((3,)),
            pltpu.SemaphoreType.DMA((3,)),
        ],
        compiler_params=pltpu.CompilerParams(collective_id=0),
    )(x, router, W1, W2)


# device time: 27908 ns/iter; 1.0957x vs baseline; 1.0957x over previous
import jax
import jax.numpy as jnp
from jax import lax
from jax.experimental import pallas as pl
from jax.experimental.pallas import tpu as pltpu


def kernel(x, router, W1, W2):
    t_loc, d = x.shape
    e_loc = W1.shape[0]
    t_half = t_loc // 2

    def body(x_ref, r_ref, w1_ref, w2_ref, out_ref,
             r_other, tok_in, c_out, c_in, c_swap, o_mine, o_swap,
             send_sems, recv_sems):
        my_x = lax.axis_index("x")
        my_y = lax.axis_index("y")
        my_z = lax.axis_index("z")
        yp = (my_x, 1 - my_y, my_z)
        zn = (my_x, my_y, 1 - my_z)

        barrier = pltpu.get_barrier_semaphore()
        for nbr in (yp, zn):
            pl.semaphore_signal(barrier, inc=1, device_id=nbr,
                                device_id_type=pl.DeviceIdType.MESH)
        pl.semaphore_wait(barrier, 2)

        rdma_r = pltpu.make_async_remote_copy(
            src_ref=r_ref, dst_ref=r_other,
            send_sem=send_sems.at[0], recv_sem=recv_sems.at[0],
            device_id=yp, device_id_type=pl.DeviceIdType.MESH)
        rdma_tok = pltpu.make_async_remote_copy(
            src_ref=x_ref.at[pl.ds(my_z * t_half, t_half)],
            dst_ref=tok_in,
            send_sem=send_sems.at[1], recv_sem=recv_sems.at[1],
            device_id=yp, device_id_type=pl.DeviceIdType.MESH)
        rdma_r.start()
        rdma_tok.start()
        rdma_r.wait_recv()
        rdma_tok.wait_recv()

        r_mine = r_ref[:, :]
        r_oth = r_other[:, :]

        def expert_partial(xh):
            g = jnp.concatenate(
                [jnp.dot(xh, r_mine, preferred_element_type=jnp.float32),
                 jnp.dot(xh, r_oth, preferred_element_type=jnp.float32)],
                axis=1)
            m1 = jnp.max(g, axis=1, keepdims=True)
            m2 = jnp.max(jnp.where(g < m1, g, -jnp.inf), axis=1,
                         keepdims=True)
            sel = g >= m2
            e = jnp.where(sel, jnp.exp(g - m1), 0.0)
            w = e / jnp.sum(e, axis=1, keepdims=True)
            acc = jnp.zeros((t_half, d), dtype=jnp.float32)
            for j in range(e_loc):
                h = jnp.maximum(
                    jnp.dot(xh, w1_ref[j], preferred_element_type=jnp.float32),
                    0.0)
                pj = jnp.dot(h, w2_ref[j], preferred_element_type=jnp.float32)
                acc = acc + pj * w[:, j:j + 1]
            return acc

        c_out[:, :] = expert_partial(tok_in[:, :])
        rdma_c = pltpu.make_async_remote_copy(
            src_ref=c_out, dst_ref=c_in,
            send_sem=send_sems.at[2], recv_sem=recv_sems.at[2],
            device_id=yp, device_id_type=pl.DeviceIdType.MESH)
        rdma_c.start()

        o_mine[:, :] = expert_partial(x_ref[pl.ds(my_z * t_half, t_half), :])
        rdma_o = pltpu.make_async_remote_copy(
            src_ref=o_mine, dst_ref=o_swap,
            send_sem=send_sems.at[3], recv_sem=recv_sems.at[3],
            device_id=zn, device_id_type=pl.DeviceIdType.MESH)
        rdma_o.start()

        rdma_c.wait_recv()
        rdma_fwd = pltpu.make_async_remote_copy(
            src_ref=c_in, dst_ref=c_swap,
            send_sem=send_sems.at[4], recv_sem=recv_sems.at[4],
            device_id=zn, device_id_type=pl.DeviceIdType.MESH)
        rdma_fwd.start()

        out_ref[pl.ds(my_z * t_half, t_half), :] = o_mine[:, :] + c_in[:, :]
        rdma_o.wait_recv()
        rdma_fwd.wait_recv()
        out_ref[pl.ds((1 - my_z) * t_half, t_half), :] = (
            o_swap[:, :] + c_swap[:, :])

        rdma_r.wait_send()
        rdma_tok.wait_send()
        rdma_c.wait_send()
        rdma_o.wait_send()
        rdma_fwd.wait_send()

    return pl.pallas_call(
        body,
        out_shape=jax.ShapeDtypeStruct((t_loc, d), jnp.float32),
        in_specs=[pl.BlockSpec(memory_space=pltpu.VMEM)] * 4,
        out_specs=pl.BlockSpec(memory_space=pltpu.VMEM),
        scratch_shapes=[
            pltpu.VMEM(router.shape, jnp.float32),
            pltpu.VMEM((t_half, d), jnp.float32),
            pltpu.VMEM((t_half, d), jnp.float32),
            pltpu.VMEM((t_half, d), jnp.float32),
            pltpu.VMEM((t_half, d), jnp.float32),
            pltpu.VMEM((t_half, d), jnp.float32),
            pltpu.VMEM((t_half, d), jnp.float32),
            pltpu.SemaphoreType.DMA((5,)),
            pltpu.SemaphoreType.DMA((5,)),
        ],
        compiler_params=pltpu.CompilerParams(collective_id=0),
    )(x, router, W1, W2)
